# Initial kernel scaffold; baseline (speedup 1.0000x reference)
import jax
import jax.numpy as jnp
from jax.experimental import pallas as pl


def kernel(h, x, edge_index, W_e1, b_e1, W_e2, b_e2, ln_g, ln_b, W_h1, b_h1, W_h2, b_h2, W_x, b_x, W_g, b_g):
    raise NotImplementedError("write your pallas kernel here")


# speedup vs baseline: 3.4456x; 3.4456x over previous
import functools

import jax
import jax.numpy as jnp
from jax import lax
from jax.experimental import pallas as pl
from jax.experimental.pallas import tpu as pltpu
from jax.experimental.pallas import tpu_sc as plsc

NC = 2
NS = 16
NW = NC * NS
C = 80


def _k1_body(h_ref, w1a_ref, w1b_ref, be1_ref, ps_ref, pd_ref):
    hb = h_ref[...]
    ps_ref[...] = jnp.dot(hb, w1a_ref[...],
                          preferred_element_type=jnp.float32) + be1_ref[...]
    pd_ref[...] = jnp.dot(hb, w1b_ref[...],
                          preferred_element_type=jnp.float32)


def _k1(h, w1a, w1b, be1):
    n, d = h.shape
    hdim = w1a.shape[1]
    bn = 2000
    grid = (n // bn,)
    return pl.pallas_call(
        _k1_body,
        grid=grid,
        in_specs=[
            pl.BlockSpec((bn, d), lambda i: (i, 0)),
            pl.BlockSpec((d, hdim), lambda i: (0, 0)),
            pl.BlockSpec((d, hdim), lambda i: (0, 0)),
            pl.BlockSpec((1, hdim), lambda i: (0, 0)),
        ],
        out_specs=[
            pl.BlockSpec((bn, hdim), lambda i: (i, 0)),
            pl.BlockSpec((bn, hdim), lambda i: (i, 0)),
        ],
        out_shape=[
            jax.ShapeDtypeStruct((n, hdim), jnp.float32),
            jax.ShapeDtypeStruct((n, hdim), jnp.float32),
        ],
    )(h, w1a, w1b, be1)


def _k2(psrc, pdst, xflat, src, dst):
    n, hdim = psrc.shape
    e = src.shape[0]
    ew = e // NW
    nchunk = ew // C

    mesh = plsc.VectorSubcoreMesh(core_axis_name="c", subcore_axis_name="s",
                                  num_cores=NC, num_subcores=NS)

    @functools.partial(
        pl.kernel,
        out_type=(jax.ShapeDtypeStruct((e, hdim), jnp.float32),
                  jax.ShapeDtypeStruct((e, 4), jnp.float32)),
        mesh=mesh,
        scratch_types=[
            pltpu.VMEM((C,), jnp.int32),
            pltpu.VMEM((C,), jnp.int32),
            pltpu.VMEM((C, hdim), jnp.float32),
            pltpu.VMEM((C, 4), jnp.float32),
            pltpu.VMEM((4, n), jnp.float32),
        ],
        compiler_params=pltpu.CompilerParams(needs_layout_passes=False, use_tc_tiling_on_sc=False),
    )
    def k2(psrc_hbm, pdst_hbm, xflat_hbm, src_hbm, dst_hbm,
           pre_hbm, aux_hbm, sidx_v, didx_v, rows_v, aux_v, xtab_v):
        cid = lax.axis_index("c")
        sid = lax.axis_index("s")
        wid = cid * NS + sid
        pltpu.sync_copy(xflat_hbm, xtab_v)
        base0 = wid * ew

        @pl.loop(0, nchunk)
        def _chunk(j):
            base = base0 + j * C
            pltpu.sync_copy(src_hbm.at[pl.ds(base, C)], sidx_v)
            pltpu.sync_copy(dst_hbm.at[pl.ds(base, C)], didx_v)
            pltpu.sync_copy(psrc_hbm.at[sidx_v], rows_v)
            pltpu.sync_copy(pdst_hbm.at[didx_v], rows_v, add=True)

            @pl.loop(0, C // 16)
            def _grp(g):
                sv = sidx_v[pl.ds(g * 16, 16)]
                dv = didx_v[pl.ds(g * 16, 16)]
                el = lax.iota(jnp.int32, 16) + g * 16
                for comp in range(3):
                    cvec = jnp.full((16,), comp, jnp.int32)
                    xs = plsc.load_gather(xtab_v, [cvec, sv])
                    xd = plsc.load_gather(xtab_v, [cvec, dv])
                    plsc.store_scatter(
                        aux_v, [el, jnp.full((16,), comp, jnp.int32)], xs - xd)
                plsc.store_scatter(
                    aux_v, [el, jnp.full((16,), 3, jnp.int32)],
                    jnp.zeros((16,), jnp.float32))

            pltpu.sync_copy(rows_v, pre_hbm.at[pl.ds(base, C)])
            pltpu.sync_copy(aux_v, aux_hbm.at[pl.ds(base, C)])

    return k2(psrc, pdst, xflat, src, dst)


def _k3_body(pre_ref, aux_ref, w1c_ref, w2_ref, be2_ref, lng_ref, lnb_ref,
             wg_ref, bg_ref, wx_ref, bx_ref, mg_ref, cm_ref):
    pre = pre_ref[...]
    aux = aux_ref[...]
    be = pre.shape[0]
    dsq = jnp.sum(aux * aux, axis=-1, keepdims=True)
    t = pre + dsq * w1c_ref[...]
    m1 = jax.nn.silu(t)
    z = jnp.dot(m1, w2_ref[...], preferred_element_type=jnp.float32) \
        + be2_ref[...]
    m2 = jax.nn.silu(z)
    mu = jnp.mean(m2, axis=-1, keepdims=True)
    var = jnp.mean((m2 - mu) * (m2 - mu), axis=-1, keepdims=True)
    mn = (m2 - mu) / jnp.sqrt(var + 1e-5) * lng_ref[...] + lnb_ref[...]
    alpha = jax.nn.sigmoid(
        jnp.dot(mn, wg_ref[...], preferred_element_type=jnp.float32)
        + bg_ref[...])
    mg = mn * alpha
    w = jnp.tanh(
        jnp.dot(mg, wx_ref[...], preferred_element_type=jnp.float32)
        + bx_ref[...])
    rn = aux / (jnp.sqrt(dsq) + 1e-8)
    cm = jnp.concatenate(
        [rn[:, :3] * w, jnp.ones((be, 1), jnp.float32),
         jnp.zeros((be, 4), jnp.float32)], axis=-1)
    mg_ref[...] = mg
    cm_ref[...] = cm


def _k3(pre, aux, w1c, w2, be2, lng, lnb, wg, bg, wx, bx):
    e, hdim = pre.shape
    be = 2000
    grid = (e // be,)
    full = lambda shape: pl.BlockSpec(shape, lambda i: tuple(0 for _ in shape))
    return pl.pallas_call(
        _k3_body,
        grid=grid,
        in_specs=[
            pl.BlockSpec((be, hdim), lambda i: (i, 0)),
            pl.BlockSpec((be, 4), lambda i: (i, 0)),
            full((1, hdim)),
            full((hdim, hdim)),
            full((1, hdim)),
            full((1, hdim)),
            full((1, hdim)),
            full((hdim, 1)),
            full((1, 1)),
            full((hdim, 1)),
            full((1, 1)),
        ],
        out_specs=[
            pl.BlockSpec((be, hdim), lambda i: (i, 0)),
            pl.BlockSpec((be, 8), lambda i: (i, 0)),
        ],
        out_shape=[
            jax.ShapeDtypeStruct((e, hdim), jnp.float32),
            jax.ShapeDtypeStruct((e, 8), jnp.float32),
        ],
    )(pre, aux, w1c, w2, be2, lng, lnb, wg, bg, wx, bx)


def _k4(mg, cm, dst, n):
    e, hdim = mg.shape
    ew = e // NW
    nchunk = ew // C
    npt = n // NS
    zr = 125

    mesh = plsc.VectorSubcoreMesh(core_axis_name="c", subcore_axis_name="s",
                                  num_cores=NC, num_subcores=NS)

    @functools.partial(
        pl.kernel,
        out_type=(jax.ShapeDtypeStruct((NC, n, hdim), jnp.float32),
                  jax.ShapeDtypeStruct((NC, n, 8), jnp.float32)),
        mesh=mesh,
        scratch_types=[
            pltpu.VMEM((1, C), jnp.int32),
            pltpu.VMEM((C, hdim), jnp.float32),
            pltpu.VMEM((C, 8), jnp.float32),
            pltpu.VMEM((zr, hdim), jnp.float32),
            pltpu.VMEM((npt, 8), jnp.float32),
            pltpu.VMEM_SHARED((n, hdim), jnp.float32),
            pltpu.VMEM_SHARED((n, 8), jnp.float32),
        ],
        compiler_params=pltpu.CompilerParams(needs_layout_passes=False, use_tc_tiling_on_sc=False),
    )
    def k4(mg_hbm, cm_hbm, dst_hbm, outh_hbm, outx_hbm,
           didx_v, mg_v, cm_v, zb64_v, zb8_v, aggh_s, aggx_s):
        cid = lax.axis_index("c")
        sid = lax.axis_index("s")
        wid = cid * NS + sid

        zeros16 = jnp.zeros((16,), jnp.float32)

        @pl.loop(0, zr)
        def _z64(r):
            for k in range(hdim // 16):
                zb64_v[r, pl.ds(k * 16, 16)] = zeros16

        @pl.loop(0, npt * 8 // 16)
        def _z8(k):
            q = lax.iota(jnp.int32, 16) + k * 16
            plsc.store_scatter(zb8_v, [q >> 3, q & 7], zeros16)

        @pl.loop(0, npt // zr)
        def _init(k):
            pltpu.sync_copy(zb64_v, aggh_s.at[pl.ds(sid * npt + k * zr, zr)])
        pltpu.sync_copy(zb8_v, aggx_s.at[pl.ds(sid * npt, npt)])

        plsc.subcore_barrier()

        base0 = wid * ew

        @pl.loop(0, nchunk)
        def _chunk(j):
            base = base0 + j * C
            pltpu.sync_copy(dst_hbm.at[pl.ds(base, C)], didx_v.at[0])
            pltpu.sync_copy(mg_hbm.at[pl.ds(base, C)], mg_v)
            pltpu.sync_copy(cm_hbm.at[pl.ds(base, C)], cm_v)
            pltpu.sync_copy(mg_v, aggh_s.at[didx_v.at[0]], add=True)
            pltpu.sync_copy(cm_v, aggx_s.at[didx_v.at[0]], add=True)

        plsc.subcore_barrier()

        pltpu.sync_copy(aggh_s.at[pl.ds(sid * npt, npt)],
                        outh_hbm.at[cid, pl.ds(sid * npt, npt)])
        pltpu.sync_copy(aggx_s.at[pl.ds(sid * npt, npt)],
                        outx_hbm.at[cid, pl.ds(sid * npt, npt)])

    return k4(mg, cm, dst)


def _k5_body(h_ref, x_ref, ph_ref, px_ref, wh1a_ref, wh1b_ref, bh1_ref,
             wh2_ref, bh2_ref, hn_ref, xn_ref):
    hb = h_ref[...]
    aggh = ph_ref[0] + ph_ref[1]
    ax = px_ref[0] + px_ref[1]
    deg = jnp.maximum(ax[:, 3:4], 1.0)
    u = jnp.dot(hb, wh1a_ref[...], preferred_element_type=jnp.float32) \
        + jnp.dot(aggh, wh1b_ref[...], preferred_element_type=jnp.float32) \
        + bh1_ref[...]
    s = jax.nn.silu(u)
    hn_ref[...] = hb + jnp.dot(s, wh2_ref[...],
                               preferred_element_type=jnp.float32) \
        + bh2_ref[...]
    xn_ref[...] = x_ref[...] + ax[:, :3] / deg


def _k5(h, x, ph, px, wh1a, wh1b, bh1, wh2, bh2):
    n, d = h.shape
    hdim = wh1b.shape[0]
    bn = 2000
    grid = (n // bn,)
    full = lambda shape: pl.BlockSpec(shape, lambda i: tuple(0 for _ in shape))
    return pl.pallas_call(
        _k5_body,
        grid=grid,
        in_specs=[
            pl.BlockSpec((bn, d), lambda i: (i, 0)),
            pl.BlockSpec((bn, 3), lambda i: (i, 0)),
            pl.BlockSpec((NC, bn, hdim), lambda i: (0, i, 0)),
            pl.BlockSpec((NC, bn, 8), lambda i: (0, i, 0)),
            full((d, hdim)),
            full((hdim, hdim)),
            full((1, hdim)),
            full((hdim, d)),
            full((1, d)),
        ],
        out_specs=[
            pl.BlockSpec((bn, d), lambda i: (i, 0)),
            pl.BlockSpec((bn, 3), lambda i: (i, 0)),
        ],
        out_shape=[
            jax.ShapeDtypeStruct((n, d), jnp.float32),
            jax.ShapeDtypeStruct((n, 3), jnp.float32),
        ],
    )(h, x, ph, px, wh1a, wh1b, bh1, wh2, bh2)


def kernel(h, x, edge_index, W_e1, b_e1, W_e2, b_e2, ln_g, ln_b,
           W_h1, b_h1, W_h2, b_h2, W_x, b_x, W_g, b_g):
    n, d = h.shape
    hdim = W_e2.shape[0]
    e = edge_index.shape[1]
    assert e % (NW * C) == 0 and n % (NS * 125) == 0

    src = edge_index[0]
    dst = edge_index[1]
    w1a = W_e1[:d]
    w1b = W_e1[d:2 * d]
    w1c = W_e1[2 * d].reshape(1, hdim)
    xflat = jnp.concatenate(
        [x.T, jnp.zeros((1, n), jnp.float32)], axis=0)

    psrc, pdst = _k1(h, w1a, w1b, b_e1.reshape(1, hdim))
    pre, aux = _k2(psrc, pdst, xflat, src, dst)
    mg, cm = _k3(pre, aux, w1c, W_e2, b_e2.reshape(1, hdim),
                 ln_g.reshape(1, hdim), ln_b.reshape(1, hdim),
                 W_g, b_g.reshape(1, 1), W_x, b_x.reshape(1, 1))
    ph, px = _k4(mg, cm, dst, n)
    hn, xn = _k5(h, x, ph, px, W_h1[:d], W_h1[d:], b_h1.reshape(1, hdim),
                 W_h2, b_h2.reshape(1, d))
    return (hn, xn)
